# baseline (speedup 1.0000x reference)
import jax
import jax.numpy as jnp
from jax import lax
from jax.experimental import pallas as pl
from jax.experimental.pallas import tpu as pltpu
from jax.experimental.pallas import tpu_sc as plsc

_HEADS = 16
_VOCAB = 32
_NC, _NS = 2, 16
_NW = _NC * _NS
_CHUNK = 2048
_LANES = 16


def _make_lookup(n_idx):
  assert n_idx % (_NW * _CHUNK) == 0
  per_w = n_idx // _NW
  n_chunks = per_w // _CHUNK
  assert n_chunks % 2 == 0

  def body(table_hbm, idx_hbm, out_hbm, table_v, idx_v, rows_v,
           sem_i0, sem_i1, sem_o0, sem_o1):
    c = lax.axis_index("c")
    s = lax.axis_index("s")
    w = s * _NC + c
    idx0 = w * per_w

    sems_i = (sem_i0, sem_i1)
    sems_o = (sem_o0, sem_o1)

    def in_copy(g, b):
      return pltpu.make_async_copy(
          idx_hbm.at[pl.ds(idx0 + g * _CHUNK, _CHUNK)], idx_v.at[b],
          sems_i[b])

    def out_copy(g, b):
      return pltpu.make_async_copy(
          rows_v.at[b],
          out_hbm.at[pl.ds((idx0 + g * _CHUNK) * _HEADS, _CHUNK * _HEADS)],
          sems_o[b])

    pltpu.sync_copy(table_hbm, table_v)
    in_copy(0, 0).start()

    @pl.loop(0, n_chunks, step=2)
    def _pair(g0):
      for b in range(2):
        g = g0 + b

        @pl.when(g + 1 < n_chunks)
        def _prefetch():
          in_copy(g + 1, 1 - b).start()

        in_copy(g, b).wait()

        @pl.when(g >= 2)
        def _drain():
          out_copy(g - 2, b).wait()

        idx_vb = idx_v.at[b]
        rows_vb = rows_v.at[b]

        lanes16 = jnp.arange(_LANES, dtype=jnp.int32) * _HEADS

        @pl.loop(0, _CHUNK // _LANES)
        def _grp(j):
          base = j * _LANES
          iv16 = idx_vb[pl.ds(base, _LANES)] * _HEADS
          dst = lanes16 + base * _HEADS
          for h in range(_HEADS):
            vals = plsc.load_gather(table_v, [iv16 + h])
            plsc.store_scatter(rows_vb, [dst + h], vals)

        out_copy(g, b).start()

    out_copy(n_chunks - 2, 0).wait()
    out_copy(n_chunks - 1, 1).wait()

  return pl.kernel(
      body,
      out_type=jax.ShapeDtypeStruct((n_idx * _HEADS,), jnp.float32),
      mesh=plsc.VectorSubcoreMesh(core_axis_name="c", subcore_axis_name="s",
                                  num_cores=_NC, num_subcores=_NS),
      scratch_types=[
          pltpu.VMEM((_VOCAB * _HEADS,), jnp.float32),
          pltpu.VMEM((2, _CHUNK), jnp.int32),
          pltpu.VMEM((2, _CHUNK * _HEADS), jnp.float32),
          pltpu.SemaphoreType.DMA,
          pltpu.SemaphoreType.DMA,
          pltpu.SemaphoreType.DMA,
          pltpu.SemaphoreType.DMA,
      ],
      compiler_params=pltpu.CompilerParams(use_tc_tiling_on_sc=False,
                                           needs_layout_passes=False),
  )


def kernel(edge_rel_pos, table):
  shape = edge_rel_pos.shape
  n_idx = edge_rel_pos.size
  idx = edge_rel_pos.reshape(n_idx).astype(jnp.int32)
  table_f = table.astype(jnp.float32).reshape(_VOCAB * _HEADS)
  out = _make_lookup(n_idx)(table_f, idx)
  return out.reshape(shape + (_HEADS,))


# speedup vs baseline: 1.6209x; 1.6209x over previous
import jax
import jax.numpy as jnp
from jax import lax
from jax.experimental import pallas as pl
from jax.experimental.pallas import tpu as pltpu
from jax.experimental.pallas import tpu_sc as plsc

_HEADS = 16
_VOCAB = 32
_NC, _NS = 2, 16
_NW = _NC * _NS
_CHUNK = 2048
_LANES = 16
_UNROLL = 4


def _make_lookup(n_idx):
  assert n_idx % (_NW * _CHUNK) == 0
  per_w = n_idx // _NW
  n_chunks = per_w // _CHUNK
  assert n_chunks % 2 == 0

  def body(table_hbm, idx_hbm, out_hbm, table_v, idx_v, rows_v,
           sem_i0, sem_i1, sem_o0, sem_o1):
    c = lax.axis_index("c")
    s = lax.axis_index("s")
    w = s * _NC + c
    idx0 = w * per_w

    sems_i = (sem_i0, sem_i1)
    sems_o = (sem_o0, sem_o1)

    def in_copy(g, b):
      return pltpu.make_async_copy(
          idx_hbm.at[pl.ds(idx0 + g * _CHUNK, _CHUNK)], idx_v.at[b],
          sems_i[b])

    def out_copy(g, b):
      return pltpu.make_async_copy(
          rows_v.at[b],
          out_hbm.at[pl.ds((idx0 + g * _CHUNK) * _HEADS, _CHUNK * _HEADS)],
          sems_o[b])

    pltpu.sync_copy(table_hbm, table_v)
    in_copy(0, 0).start()

    @pl.loop(0, n_chunks, step=2)
    def _pair(g0):
      for b in range(2):
        g = g0 + b

        @pl.when(g + 1 < n_chunks)
        def _prefetch():
          in_copy(g + 1, 1 - b).start()

        in_copy(g, b).wait()

        @pl.when(g >= 2)
        def _drain():
          out_copy(g - 2, b).wait()

        idx_vb = idx_v.at[b]
        rows_vb = rows_v.at[b]

        @plsc.parallel_loop(0, _CHUNK // _LANES, unroll=_UNROLL)
        def _grp(j):
          base = j * _LANES
          iv = idx_vb[pl.ds(base, _LANES)] * _HEADS
          for u in range(_LANES):
            row = table_v[pl.ds(iv[u], _HEADS)]
            rows_vb[pl.ds((base + u) * _HEADS, _HEADS)] = row

        out_copy(g, b).start()

    out_copy(n_chunks - 2, 0).wait()
    out_copy(n_chunks - 1, 1).wait()

  return pl.kernel(
      body,
      out_type=jax.ShapeDtypeStruct((n_idx * _HEADS,), jnp.float32),
      mesh=plsc.VectorSubcoreMesh(core_axis_name="c", subcore_axis_name="s",
                                  num_cores=_NC, num_subcores=_NS),
      scratch_types=[
          pltpu.VMEM((_VOCAB * _HEADS,), jnp.float32),
          pltpu.VMEM((2, _CHUNK), jnp.int32),
          pltpu.VMEM((2, _CHUNK * _HEADS), jnp.float32),
          pltpu.SemaphoreType.DMA,
          pltpu.SemaphoreType.DMA,
          pltpu.SemaphoreType.DMA,
          pltpu.SemaphoreType.DMA,
      ],
      compiler_params=pltpu.CompilerParams(use_tc_tiling_on_sc=False,
                                           needs_layout_passes=False),
  )


def kernel(edge_rel_pos, table):
  shape = edge_rel_pos.shape
  n_idx = edge_rel_pos.size
  idx = edge_rel_pos.reshape(n_idx).astype(jnp.int32)
  table_f = table.astype(jnp.float32).reshape(_VOCAB * _HEADS)
  out = _make_lookup(n_idx)(table_f, idx)
  return out.reshape(shape + (_HEADS,))


# speedup vs baseline: 1.6227x; 1.0011x over previous
import jax
import jax.numpy as jnp
from jax import lax
from jax.experimental import pallas as pl
from jax.experimental.pallas import tpu as pltpu
from jax.experimental.pallas import tpu_sc as plsc

_HEADS = 16
_VOCAB = 32
_NC, _NS = 2, 16
_NW = _NC * _NS
_CHUNK = 2048
_LANES = 16
_UNROLL = 4


def _make_lookup(n_idx):
  assert n_idx % (_NW * _CHUNK) == 0
  per_w = n_idx // _NW
  n_chunks = per_w // _CHUNK
  assert n_chunks % 2 == 0

  def body(table_hbm, idx_hbm, out_hbm, table_v, idx_v, rows_v,
           sem_i0, sem_i1, sem_o0, sem_o1):
    c = lax.axis_index("c")
    s = lax.axis_index("s")
    w = s * _NC + c
    idx0 = w * per_w

    sems_i = (sem_i0, sem_i1)
    sems_o = (sem_o0, sem_o1)

    def in_copy(g, b):
      return pltpu.make_async_copy(
          idx_hbm.at[pl.ds(idx0 + g * _CHUNK, _CHUNK)], idx_v.at[b],
          sems_i[b])

    def out_copy(g, b):
      return pltpu.make_async_copy(
          rows_v.at[b],
          out_hbm.at[pl.ds((idx0 + g * _CHUNK) * _HEADS, _CHUNK * _HEADS)],
          sems_o[b])

    pltpu.sync_copy(table_hbm, table_v)
    in_copy(0, 0).start()

    @pl.loop(0, n_chunks, step=2)
    def _pair(g0):
      for b in range(2):
        g = g0 + b

        @pl.when(g + 1 < n_chunks)
        def _prefetch():
          in_copy(g + 1, 1 - b).start()

        in_copy(g, b).wait()

        @pl.when(g >= 2)
        def _drain():
          out_copy(g - 2, b).wait()

        idx_vb = idx_v.at[b]
        rows_vb = rows_v.at[b]

        lanes = jnp.arange(_LANES, dtype=jnp.int32)

        @plsc.parallel_loop(0, _CHUNK // _LANES, unroll=_UNROLL)
        def _grp(j):
          base = j * _LANES
          iv = idx_vb[pl.ds(base, _LANES)] * _HEADS
          dstb = (lanes + base) * _HEADS
          for k in range(_HEADS):
            h = (lanes + k) & (_HEADS - 1)
            vals = plsc.load_gather(table_v, [iv + h])
            plsc.store_scatter(rows_vb, [dstb + h], vals)

        out_copy(g, b).start()

    out_copy(n_chunks - 2, 0).wait()
    out_copy(n_chunks - 1, 1).wait()

  return pl.kernel(
      body,
      out_type=jax.ShapeDtypeStruct((n_idx * _HEADS,), jnp.float32),
      mesh=plsc.VectorSubcoreMesh(core_axis_name="c", subcore_axis_name="s",
                                  num_cores=_NC, num_subcores=_NS),
      scratch_types=[
          pltpu.VMEM((_VOCAB * _HEADS,), jnp.float32),
          pltpu.VMEM((2, _CHUNK), jnp.int32),
          pltpu.VMEM((2, _CHUNK * _HEADS), jnp.float32),
          pltpu.SemaphoreType.DMA,
          pltpu.SemaphoreType.DMA,
          pltpu.SemaphoreType.DMA,
          pltpu.SemaphoreType.DMA,
      ],
      compiler_params=pltpu.CompilerParams(use_tc_tiling_on_sc=False,
                                           needs_layout_passes=False),
  )


def kernel(edge_rel_pos, table):
  shape = edge_rel_pos.shape
  n_idx = edge_rel_pos.size
  idx = edge_rel_pos.reshape(n_idx).astype(jnp.int32)
  table_f = table.astype(jnp.float32).reshape(_VOCAB * _HEADS)
  out = _make_lookup(n_idx)(table_f, idx)
  return out.reshape(shape + (_HEADS,))
